# Initial kernel scaffold; baseline (speedup 1.0000x reference)
import functools

import jax
import jax.numpy as jnp
from jax import lax
from jax.experimental import pallas as pl
from jax.experimental.pallas import tpu as pltpu
from jax.experimental.pallas import tpu_sc as plsc

_NC = 2
_NS = 16
_LANES = 16
_CHUNK = 128


def _mm_body(x_ref, w_ref, o_ref):
    o_ref[...] = jnp.dot(x_ref[...], w_ref[...],
                         preferred_element_type=jnp.float32)


def _mm(x, w, blk=2000):
    n, d = x.shape
    return pl.pallas_call(
        _mm_body,
        grid=(n // blk,),
        in_specs=[
            pl.BlockSpec((blk, d), lambda i: (i, 0)),
            pl.BlockSpec((d, w.shape[1]), lambda i: (0, 0)),
        ],
        out_specs=pl.BlockSpec((blk, w.shape[1]), lambda i: (i, 0)),
        out_shape=jax.ShapeDtypeStruct((n, w.shape[1]), jnp.float32),
    )(x, w)


def _fuse_mm_body(p_ref, b_ref, w_ref, o_ref):
    h = p_ref[0] + p_ref[1] + b_ref[...]
    h = jnp.where(h >= 0, h, 0.25 * h)
    o_ref[...] = jnp.dot(h, w_ref[...], preferred_element_type=jnp.float32)


def _fuse_mm(p, b, w, blk=2000):
    _, n, d = p.shape
    b2 = b.reshape(1, d)
    return pl.pallas_call(
        _fuse_mm_body,
        grid=(n // blk,),
        in_specs=[
            pl.BlockSpec((2, blk, d), lambda i: (0, i, 0)),
            pl.BlockSpec((1, d), lambda i: (0, 0)),
            pl.BlockSpec((d, w.shape[1]), lambda i: (0, 0)),
        ],
        out_specs=pl.BlockSpec((blk, w.shape[1]), lambda i: (i, 0)),
        out_shape=jax.ShapeDtypeStruct((n, w.shape[1]), jnp.float32),
    )(p, b2, w)


def _act_body(p_ref, b_ref, o_ref):
    h = p_ref[0] + p_ref[1] + b_ref[...]
    o_ref[...] = jnp.where(h >= 0, h, 0.25 * h)


def _act(p, b, blk=2000):
    _, n, d = p.shape
    b2 = b.reshape(1, d)
    return pl.pallas_call(
        _act_body,
        grid=(n // blk,),
        in_specs=[
            pl.BlockSpec((2, blk, d), lambda i: (0, i, 0)),
            pl.BlockSpec((1, d), lambda i: (0, 0)),
        ],
        out_specs=pl.BlockSpec((blk, d), lambda i: (i, 0)),
        out_shape=jax.ShapeDtypeStruct((n, d), jnp.float32),
    )(p, b2)


@functools.lru_cache(maxsize=None)
def _make_sc_agg(n, d, cpt):
    mesh = plsc.VectorSubcoreMesh(core_axis_name="c", subcore_axis_name="s",
                                  num_cores=_NC)
    rpt = n // _NS

    @functools.partial(
        pl.kernel,
        mesh=mesh,
        out_type=jax.ShapeDtypeStruct((_NC, n, d), jnp.float32),
        scratch_types=[
            pltpu.VMEM((_CHUNK,), jnp.int32),
            pltpu.VMEM((_CHUNK,), jnp.int32),
            pltpu.VMEM((_CHUNK,), jnp.float32),
            pltpu.VMEM((_CHUNK, d), jnp.float32),
            pltpu.VMEM_SHARED((n, d), jnp.float32),
            pltpu.SemaphoreType.DMA,
        ],
    )
    def sc_agg(support, col3, row3, w3, zeros, out,
               colv, rowv, wv, rows, acc, sem):
        cid = lax.axis_index("c")
        sid = lax.axis_index("s")
        pltpu.sync_copy(zeros.at[pl.ds(sid * rpt, rpt)],
                        acc.at[pl.ds(sid * rpt, rpt)])
        plsc.subcore_barrier()

        wid = cid * _NS + sid

        def chunk_body(j, carry):
            pltpu.sync_copy(col3.at[wid, j], colv)
            pltpu.sync_copy(row3.at[wid, j], rowv)
            pltpu.sync_copy(w3.at[wid, j], wv)
            pltpu.async_copy(support.at[colv], rows, sem).wait()

            def group_body(g, c2):
                for l in range(_LANES):
                    e = g * _LANES + l
                    ws = plsc.load_gather(
                        wv, [jnp.full((_LANES,), e, dtype=jnp.int32)])
                    for dp in range(d // _LANES):
                        sl = pl.ds(dp * _LANES, _LANES)
                        rows[e, sl] = rows[e, sl] * ws
                return c2

            lax.fori_loop(0, _CHUNK // _LANES, group_body, 0)
            pltpu.sync_copy(rows, acc.at[rowv], add=True)
            return carry

        lax.fori_loop(0, cpt, chunk_body, 0)
        plsc.subcore_barrier()
        pltpu.sync_copy(acc.at[pl.ds(sid * rpt, rpt)],
                        out.at[cid, pl.ds(sid * rpt, rpt)])

    return sc_agg


def kernel(x, edge_index, edge_weight, W1, b1, W2, b2, W3, b3):
    n, d = x.shape
    e = edge_weight.shape[0]
    nt = _NC * _NS
    cpt = -(-e // (_CHUNK * nt))
    ep = nt * cpt * _CHUNK
    pad = ep - e

    row = jnp.concatenate([edge_index[0], jnp.zeros((pad,), jnp.int32)])
    col = jnp.concatenate([edge_index[1], jnp.zeros((pad,), jnp.int32)])
    w = jnp.concatenate([edge_weight, jnp.zeros((pad,), jnp.float32)])
    row3 = row.reshape(nt, cpt, _CHUNK)
    col3 = col.reshape(nt, cpt, _CHUNK)
    w3 = w.reshape(nt, cpt, _CHUNK)
    zeros = jnp.zeros((n, d), jnp.float32)

    sc_agg = _make_sc_agg(n, d, cpt)

    s = _mm(x, W1)
    p = sc_agg(s, col3, row3, w3, zeros)
    s = _fuse_mm(p, b1, W2)
    p = sc_agg(s, col3, row3, w3, zeros)
    s = _fuse_mm(p, b2, W3)
    p = sc_agg(s, col3, row3, w3, zeros)
    return _act(p, b3)


# speedup vs baseline: 3.1817x; 3.1817x over previous
import functools

import jax
import jax.numpy as jnp
from jax import lax
from jax.experimental import pallas as pl
from jax.experimental.pallas import tpu as pltpu
from jax.experimental.pallas import tpu_sc as plsc

_NC = 2
_NS = 16
_LANES = 16
_CHUNK = 128


def _mm_body(x_ref, w_ref, o_ref):
    o_ref[...] = jnp.dot(x_ref[...], w_ref[...],
                         preferred_element_type=jnp.float32)


def _mm(x, w, blk=2000):
    n, d = x.shape
    return pl.pallas_call(
        _mm_body,
        grid=(n // blk,),
        in_specs=[
            pl.BlockSpec((blk, d), lambda i: (i, 0)),
            pl.BlockSpec((d, w.shape[1]), lambda i: (0, 0)),
        ],
        out_specs=pl.BlockSpec((blk, w.shape[1]), lambda i: (i, 0)),
        out_shape=jax.ShapeDtypeStruct((n, w.shape[1]), jnp.float32),
    )(x, w)


def _fuse_mm_body(p_ref, b_ref, w_ref, o_ref):
    h = p_ref[0] + p_ref[1] + b_ref[...]
    h = jnp.where(h >= 0, h, 0.25 * h)
    o_ref[...] = jnp.dot(h, w_ref[...], preferred_element_type=jnp.float32)


def _fuse_mm(p, b, w, blk=2000):
    _, n, d = p.shape
    b2 = b.reshape(1, d)
    return pl.pallas_call(
        _fuse_mm_body,
        grid=(n // blk,),
        in_specs=[
            pl.BlockSpec((2, blk, d), lambda i: (0, i, 0)),
            pl.BlockSpec((1, d), lambda i: (0, 0)),
            pl.BlockSpec((d, w.shape[1]), lambda i: (0, 0)),
        ],
        out_specs=pl.BlockSpec((blk, w.shape[1]), lambda i: (i, 0)),
        out_shape=jax.ShapeDtypeStruct((n, w.shape[1]), jnp.float32),
    )(p, b2, w)


def _act_body(p_ref, b_ref, o_ref):
    h = p_ref[0] + p_ref[1] + b_ref[...]
    o_ref[...] = jnp.where(h >= 0, h, 0.25 * h)


def _act(p, b, blk=2000):
    _, n, d = p.shape
    b2 = b.reshape(1, d)
    return pl.pallas_call(
        _act_body,
        grid=(n // blk,),
        in_specs=[
            pl.BlockSpec((2, blk, d), lambda i: (0, i, 0)),
            pl.BlockSpec((1, d), lambda i: (0, 0)),
        ],
        out_specs=pl.BlockSpec((blk, d), lambda i: (i, 0)),
        out_shape=jax.ShapeDtypeStruct((n, d), jnp.float32),
    )(p, b2)


@functools.lru_cache(maxsize=None)
def _make_sc_agg(n, d, cpt):
    mesh = plsc.VectorSubcoreMesh(core_axis_name="c", subcore_axis_name="s",
                                  num_cores=_NC)
    rpt = (n // _NS) & ~7
    tail = n - _NS * rpt

    @functools.partial(
        pl.kernel,
        mesh=mesh,
        out_type=jax.ShapeDtypeStruct((_NC, n, d), jnp.float32),
        scratch_types=[
            pltpu.VMEM((_CHUNK,), jnp.int32),
            pltpu.VMEM((_CHUNK,), jnp.int32),
            pltpu.VMEM((_CHUNK,), jnp.float32),
            pltpu.VMEM((_CHUNK, d), jnp.float32),
            pltpu.VMEM_SHARED((n, d), jnp.float32),
            pltpu.SemaphoreType.DMA,
        ],
    )
    def sc_agg(support, col3, row3, w3, zeros, out,
               colv, rowv, wv, rows, acc, sem):
        cid = lax.axis_index("c")
        sid = lax.axis_index("s")
        pltpu.sync_copy(zeros.at[pl.ds(sid * rpt, rpt)],
                        acc.at[pl.ds(sid * rpt, rpt)])
        if tail:
            @pl.when(sid == 0)
            def _zero_tail():
                pltpu.sync_copy(zeros.at[pl.ds(_NS * rpt, tail)],
                                acc.at[pl.ds(_NS * rpt, tail)])
        plsc.subcore_barrier()

        wid = cid * _NS + sid

        def chunk_body(j, carry):
            pltpu.sync_copy(col3.at[wid, j], colv)
            pltpu.sync_copy(row3.at[wid, j], rowv)
            pltpu.sync_copy(w3.at[wid, j], wv)
            pltpu.async_copy(support.at[colv], rows, sem).wait()

            def group_body(g, c2):
                wg = wv[pl.ds(g * _LANES, _LANES)]
                for l in range(_LANES):
                    e = g * _LANES + l
                    ws = wg[l]
                    for dp in range(d // _LANES):
                        sl = pl.ds(dp * _LANES, _LANES)
                        rows[e, sl] = rows[e, sl] * ws
                return c2

            lax.fori_loop(0, _CHUNK // _LANES, group_body, 0)
            pltpu.sync_copy(rows, acc.at[rowv], add=True)
            return carry

        lax.fori_loop(0, cpt, chunk_body, 0)
        plsc.subcore_barrier()
        pltpu.sync_copy(acc.at[pl.ds(sid * rpt, rpt)],
                        out.at[cid, pl.ds(sid * rpt, rpt)])
        if tail:
            @pl.when(sid == 0)
            def _write_tail():
                pltpu.sync_copy(acc.at[pl.ds(_NS * rpt, tail)],
                                out.at[cid, pl.ds(_NS * rpt, tail)])

    return sc_agg


def kernel(x, edge_index, edge_weight, W1, b1, W2, b2, W3, b3):
    n, d = x.shape
    e = edge_weight.shape[0]
    nt = _NC * _NS
    cpt = -(-e // (_CHUNK * nt))
    ep = nt * cpt * _CHUNK
    pad = ep - e

    row = jnp.concatenate([edge_index[0], jnp.zeros((pad,), jnp.int32)])
    col = jnp.concatenate([edge_index[1], jnp.zeros((pad,), jnp.int32)])
    w = jnp.concatenate([edge_weight, jnp.zeros((pad,), jnp.float32)])
    row3 = row.reshape(nt, cpt, _CHUNK)
    col3 = col.reshape(nt, cpt, _CHUNK)
    w3 = w.reshape(nt, cpt, _CHUNK)
    zeros = jnp.zeros((n, d), jnp.float32)

    sc_agg = _make_sc_agg(n, d, cpt)

    s = _mm(x, W1)
    p = sc_agg(s, col3, row3, w3, zeros)
    s = _fuse_mm(p, b1, W2)
    p = sc_agg(s, col3, row3, w3, zeros)
    s = _fuse_mm(p, b2, W3)
    p = sc_agg(s, col3, row3, w3, zeros)
    return _act(p, b3)
